# baseline (speedup 1.0000x reference)
import functools

import jax
import jax.numpy as jnp
from jax import lax
from jax.experimental import pallas as pl
from jax.experimental.pallas import tpu as pltpu
from jax.experimental.pallas import tpu_sc as plsc

N = 10000
E = 320000
D = 128
G = 128
C = 10

NCORES = 2
NSUB = 16
NW = NCORES * NSUB
EPW = E // NW
CHUNK = 40
NCHUNK = EPW // CHUNK
DEPTH = 5
RPT = 624
REM = N - NSUB * RPT

BLK = 1000
NBLK = N // BLK

_PREC = lax.Precision.DEFAULT


def _sc_body(h_hbm, eidx_hbm, zeros_hbm, out_hbm, *refs):
    c = lax.axis_index("c")
    s = lax.axis_index("s")
    wid = c * NSUB + s
    r0 = s * RPT

    ib = refs[0:2 * DEPTH]
    rows = refs[2 * DEPTH:3 * DEPTH]
    acc = refs[3 * DEPTH]
    gsem = refs[3 * DEPTH + 1:4 * DEPTH + 1]
    ssem = refs[4 * DEPTH + 1:5 * DEPTH + 1]
    isem = refs[5 * DEPTH + 1:7 * DEPTH + 1]

    def istart(i, q):
        pltpu.async_copy(eidx_hbm.at[wid].at[i], ib[q], isem[q])

    def iwait(q):
        pltpu.make_async_copy(eidx_hbm.at[0].at[0], ib[q], isem[q]).wait()

    def gstart(q, k):
        pltpu.async_copy(h_hbm.at[ib[q].at[0]], rows[k], gsem[k])

    def gwait(k):
        pltpu.make_async_copy(h_hbm.at[pl.ds(0, CHUNK)], rows[k],
                              gsem[k]).wait()

    def sstart(k, q):
        pltpu.async_copy(rows[k], acc.at[ib[q].at[1]], ssem[k], add=True)

    def swait(k):
        pltpu.make_async_copy(h_hbm.at[pl.ds(0, CHUNK)], rows[k],
                              ssem[k]).wait()

    for q in range(2 * DEPTH):
        istart(q, q)

    @pl.when(c == 0)
    def _():
        pltpu.sync_copy(h_hbm.at[pl.ds(r0, RPT)], acc.at[pl.ds(r0, RPT)])

        @pl.when(s == NSUB - 1)
        def _():
            pltpu.sync_copy(h_hbm.at[pl.ds(NSUB * RPT, REM)],
                            acc.at[pl.ds(NSUB * RPT, REM)])

    @pl.when(c == 1)
    def _():
        pltpu.sync_copy(zeros_hbm.at[pl.ds(r0, RPT)], acc.at[pl.ds(r0, RPT)])

        @pl.when(s == NSUB - 1)
        def _():
            pltpu.sync_copy(zeros_hbm.at[pl.ds(NSUB * RPT, REM)],
                            acc.at[pl.ds(NSUB * RPT, REM)])

    for k in range(DEPTH):
        iwait(k)
        gstart(k, k)

    plsc.subcore_barrier()

    def body(j, carry):
        i = 2 * DEPTH * j

        for k in range(DEPTH):
            gwait(k)
            sstart(k, k)
        for k in range(DEPTH):
            swait(k)
            istart(i + 2 * DEPTH + k, k)
            iwait(DEPTH + k)
            gstart(DEPTH + k, k)

        for k in range(DEPTH):
            gwait(k)
            sstart(k, DEPTH + k)
        for k in range(DEPTH):
            swait(k)
            istart(i + 3 * DEPTH + k, DEPTH + k)
            iwait(k)
            gstart(k, k)

        return carry

    lax.fori_loop(0, NCHUNK // (2 * DEPTH) - 1, body, 0)

    for k in range(DEPTH):
        gwait(k)
        sstart(k, k)
    for k in range(DEPTH):
        swait(k)
        iwait(DEPTH + k)
        gstart(DEPTH + k, k)
    for k in range(DEPTH):
        gwait(k)
        sstart(k, DEPTH + k)
    for k in range(DEPTH):
        swait(k)

    plsc.subcore_barrier()

    pltpu.sync_copy(acc.at[pl.ds(r0, RPT)],
                    out_hbm.at[pl.ds(c * N + r0, RPT)])

    @pl.when(s == NSUB - 1)
    def _():
        pltpu.sync_copy(acc.at[pl.ds(NSUB * RPT, REM)],
                        out_hbm.at[pl.ds(c * N + NSUB * RPT, REM)])


def _sc_segment_sum(h, eidx, zeros):
    mesh = plsc.VectorSubcoreMesh(core_axis_name="c", subcore_axis_name="s")
    f = pl.kernel(
        _sc_body,
        out_type=jax.ShapeDtypeStruct((2 * N, D), jnp.float32),
        mesh=mesh,
        scratch_types=(
            [pltpu.VMEM((2, CHUNK), jnp.int32)] * (2 * DEPTH)
            + [pltpu.VMEM((CHUNK, D), jnp.float32)] * DEPTH
            + [pltpu.VMEM_SHARED((N, D), jnp.float32)]
            + [pltpu.SemaphoreType.DMA] * (4 * DEPTH)
        ),
    )
    return f(h, eidx, zeros)


def _mlp_phase0(i, p_ref, wa_ref, ba_ref, wb_ref, bb_ref,
                rbuf_ref, a0_ref, a1_ref):
    hin = p_ref[0] + p_ref[1]
    t = jnp.maximum(
        jnp.dot(hin, wa_ref[...], preferred_element_type=jnp.float32,
                precision=_PREC) + ba_ref[...], 0.0)
    r = jnp.maximum(
        jnp.dot(t, wb_ref[...], preferred_element_type=jnp.float32,
                precision=_PREC) + bb_ref[...], 0.0)
    rbuf_ref[pl.ds(i * BLK, BLK), :] = r

    @pl.when(i == 0)
    def _():
        a0_ref[...] = jnp.zeros_like(a0_ref)
        a1_ref[...] = jnp.zeros_like(a1_ref)

    a0_ref[...] += jnp.sum(r, axis=0, keepdims=True)
    a1_ref[...] += jnp.sum(r * r, axis=0, keepdims=True)


def _bn_block(j, rbuf_ref, a0_ref, a1_ref, g_ref, be_ref):
    m = a0_ref[...] * (1.0 / N)
    v = a1_ref[...] * (1.0 / N) - m * m
    inv = lax.rsqrt(v + 1e-5) * g_ref[...]
    return (rbuf_ref[pl.ds(j * BLK, BLK), :] - m) * inv + be_ref[...]


def _layer_body(p_ref, wa_ref, ba_ref, wb_ref, bb_ref,
                g_ref, be_ref, o_ref, rbuf_ref, a0_ref, a1_ref):
    i = pl.program_id(0)

    @pl.when(i < NBLK)
    def _():
        _mlp_phase0(i, p_ref, wa_ref, ba_ref, wb_ref,
                    bb_ref, rbuf_ref, a0_ref, a1_ref)

    @pl.when(i >= NBLK)
    def _():
        o_ref[...] = _bn_block(i - NBLK, rbuf_ref, a0_ref, a1_ref,
                               g_ref, be_ref)


def _tc_layer(parts, wa, ba, wb, bb, g, be):
    full = lambda i: (0, 0)
    return pl.pallas_call(
        _layer_body,
        grid=(2 * NBLK,),
        in_specs=[
            pl.BlockSpec((2, BLK, D),
                         lambda i: (0, jnp.minimum(i, NBLK - 1), 0)),
            pl.BlockSpec((D, D), full),
            pl.BlockSpec((1, D), full),
            pl.BlockSpec((D, D), full),
            pl.BlockSpec((1, D), full),
            pl.BlockSpec((1, D), full),
            pl.BlockSpec((1, D), full),
        ],
        out_specs=pl.BlockSpec((BLK, D), lambda i: (jnp.maximum(i - NBLK,
                                                                0), 0)),
        out_shape=jax.ShapeDtypeStruct((N, D), jnp.float32),
        scratch_shapes=[
            pltpu.VMEM((N, D), jnp.float32),
            pltpu.VMEM((1, D), jnp.float32),
            pltpu.VMEM((1, D), jnp.float32),
        ],
        compiler_params=pltpu.CompilerParams(
            dimension_semantics=("arbitrary",)),
    )(parts.reshape(2, N, D), wa, ba, wb, bb, g, be)


def _final_body(p_ref, b_ref, wa_ref, ba_ref, wb_ref,
                bb_ref, g_ref, be_ref, wf1_ref, bf1_ref, wf2_ref, bf2_ref,
                o_ref, rbuf_ref, a0_ref, a1_ref, pacc_ref):
    i = pl.program_id(0)

    @pl.when(i < NBLK)
    def _():
        _mlp_phase0(i, p_ref, wa_ref, ba_ref, wb_ref,
                    bb_ref, rbuf_ref, a0_ref, a1_ref)

    @pl.when(i >= NBLK)
    def _():
        j = i - NBLK
        hn = _bn_block(j, rbuf_ref, a0_ref, a1_ref, g_ref, be_ref)
        gids = lax.broadcasted_iota(jnp.int32, (1, G), 1).astype(jnp.float32)
        onehot = (b_ref[...] == gids).astype(jnp.float32)

        @pl.when(j == 0)
        def _():
            pacc_ref[...] = jnp.zeros_like(pacc_ref)

        pacc_ref[...] += lax.dot_general(
            onehot, hn, (((0,), (0,)), ((), ())),
            preferred_element_type=jnp.float32, precision=_PREC)

        @pl.when(j == NBLK - 1)
        def _():
            p = jnp.maximum(
                jnp.dot(pacc_ref[...], wf1_ref[...],
                        preferred_element_type=jnp.float32,
                        precision=_PREC) + bf1_ref[...], 0.0)
            o_ref[...] = jnp.dot(p, wf2_ref[...],
                                 preferred_element_type=jnp.float32,
                                 precision=_PREC) + bf2_ref[...]


def _tc_final(parts, batchf, wa, ba, wb, bb, g, be, wf1, bf1, wf2, bf2):
    full = lambda i: (0, 0)
    return pl.pallas_call(
        _final_body,
        grid=(2 * NBLK,),
        in_specs=[
            pl.BlockSpec((2, BLK, D),
                         lambda i: (0, jnp.minimum(i, NBLK - 1), 0)),
            pl.BlockSpec((BLK, 1), lambda i: (jnp.maximum(i - NBLK, 0), 0)),
            pl.BlockSpec((D, D), full),
            pl.BlockSpec((1, D), full),
            pl.BlockSpec((D, D), full),
            pl.BlockSpec((1, D), full),
            pl.BlockSpec((1, D), full),
            pl.BlockSpec((1, D), full),
            pl.BlockSpec((D, D), full),
            pl.BlockSpec((1, D), full),
            pl.BlockSpec((D, C), full),
            pl.BlockSpec((1, C), full),
        ],
        out_specs=pl.BlockSpec((G, C), full),
        out_shape=jax.ShapeDtypeStruct((G, C), jnp.float32),
        scratch_shapes=[
            pltpu.VMEM((N, D), jnp.float32),
            pltpu.VMEM((1, D), jnp.float32),
            pltpu.VMEM((1, D), jnp.float32),
            pltpu.VMEM((G, D), jnp.float32),
        ],
        compiler_params=pltpu.CompilerParams(
            dimension_semantics=("arbitrary",)),
    )(parts.reshape(2, N, D), batchf, wa, ba, wb, bb, g, be,
      wf1, bf1, wf2, bf2)


def _layer(h, eidx, zeros, wa, ba, wb, bb, g, be):
    parts = _sc_segment_sum(h, eidx, zeros)
    return _tc_layer(parts, wa, ba, wb, bb, g, be)


def kernel(x, edge_index, batch, w1a, b1a, w1b, b1b, g1, be1,
           w2a, b2a, w2b, b2b, g2, be2, w3a, b3a, w3b, b3b, g3, be3,
           wf1, bf1, wf2, bf2):
    src = edge_index[0].reshape(NW, NCHUNK, 1, CHUNK)
    dst = edge_index[1].reshape(NW, NCHUNK, 1, CHUNK)
    eidx = jnp.concatenate([src, dst], axis=2)
    zeros = jnp.zeros((N, D), jnp.float32)
    row = lambda v: v.reshape(1, -1)

    h = _layer(x, eidx, zeros, w1a, row(b1a), w1b, row(b1b),
               row(g1), row(be1))
    h = _layer(h, eidx, zeros, w2a, row(b2a), w2b, row(b2b),
               row(g2), row(be2))

    batchf = batch.astype(jnp.float32).reshape(N, 1)
    parts3 = _sc_segment_sum(h, eidx, zeros)
    return _tc_final(parts3, batchf, w3a, row(b3a), w3b, row(b3b),
                     row(g3), row(be3), wf1, row(bf1), wf2, row(bf2))


# speedup vs baseline: 1.0766x; 1.0766x over previous
import functools

import jax
import jax.numpy as jnp
from jax import lax
from jax.experimental import pallas as pl
from jax.experimental.pallas import tpu as pltpu
from jax.experimental.pallas import tpu_sc as plsc

N = 10000
E = 320000
D = 128
G = 128
C = 10

NCORES = 2
NSUB = 16
NW = NCORES * NSUB
EPW = E // NW
CHUNK = 40
NCHUNK = EPW // CHUNK
DEPTH = 5
RPT = 624
REM = N - NSUB * RPT

BLK = 1000
NBLK = N // BLK

_PREC = lax.Precision.DEFAULT


def _sc_body(h_hbm, src_hbm, dst_hbm, zeros_hbm, out_hbm, *refs):
    c = lax.axis_index("c")
    s = lax.axis_index("s")
    wid = c * NSUB + s
    r0 = s * RPT
    base = wid * EPW

    sib = refs[0:DEPTH]
    dib = refs[DEPTH:2 * DEPTH]
    rows = refs[2 * DEPTH:3 * DEPTH]
    acc = refs[3 * DEPTH]
    gsem = refs[3 * DEPTH + 1:4 * DEPTH + 1]
    ssem = refs[4 * DEPTH + 1:5 * DEPTH + 1]
    sisem = refs[5 * DEPTH + 1:6 * DEPTH + 1]
    disem = refs[6 * DEPTH + 1:7 * DEPTH + 1]

    def si_start(i, k):
        pltpu.async_copy(src_hbm.at[pl.ds(base + i * CHUNK, CHUNK)],
                         sib[k], sisem[k])

    def si_wait(k):
        pltpu.make_async_copy(src_hbm.at[pl.ds(0, CHUNK)], sib[k],
                              sisem[k]).wait()

    def di_start(i, k):
        pltpu.async_copy(dst_hbm.at[pl.ds(base + i * CHUNK, CHUNK)],
                         dib[k], disem[k])

    def di_wait(k):
        pltpu.make_async_copy(dst_hbm.at[pl.ds(0, CHUNK)], dib[k],
                              disem[k]).wait()

    def gstart(k):
        pltpu.async_copy(h_hbm.at[sib[k]], rows[k], gsem[k])

    def gwait(k):
        pltpu.make_async_copy(h_hbm.at[pl.ds(0, CHUNK)], rows[k],
                              gsem[k]).wait()

    def sstart(k):
        pltpu.async_copy(rows[k], acc.at[dib[k]], ssem[k], add=True)

    def swait(k):
        pltpu.make_async_copy(h_hbm.at[pl.ds(0, CHUNK)], rows[k],
                              ssem[k]).wait()

    for k in range(DEPTH):
        si_start(k, k)
        di_start(k, k)

    @pl.when(c == 0)
    def _():
        pltpu.sync_copy(h_hbm.at[pl.ds(r0, RPT)], acc.at[pl.ds(r0, RPT)])

        @pl.when(s == NSUB - 1)
        def _():
            pltpu.sync_copy(h_hbm.at[pl.ds(NSUB * RPT, REM)],
                            acc.at[pl.ds(NSUB * RPT, REM)])

    @pl.when(c == 1)
    def _():
        pltpu.sync_copy(zeros_hbm.at[pl.ds(r0, RPT)], acc.at[pl.ds(r0, RPT)])

        @pl.when(s == NSUB - 1)
        def _():
            pltpu.sync_copy(zeros_hbm.at[pl.ds(NSUB * RPT, REM)],
                            acc.at[pl.ds(NSUB * RPT, REM)])

    for k in range(DEPTH):
        si_wait(k)
        gstart(k)

    plsc.subcore_barrier()

    def body(j, carry):
        i = DEPTH * j

        for k in range(DEPTH):
            gwait(k)
            si_start(i + k + DEPTH, k)
            di_wait(k)
            sstart(k)

        for k in range(DEPTH):
            swait(k)
            di_start(i + k + DEPTH, k)
            si_wait(k)
            gstart(k)

        return carry

    lax.fori_loop(0, NCHUNK // DEPTH - 1, body, 0)

    for k in range(DEPTH):
        gwait(k)
        di_wait(k)
        sstart(k)
    for k in range(DEPTH):
        swait(k)

    plsc.subcore_barrier()

    pltpu.sync_copy(acc.at[pl.ds(r0, RPT)],
                    out_hbm.at[pl.ds(c * N + r0, RPT)])

    @pl.when(s == NSUB - 1)
    def _():
        pltpu.sync_copy(acc.at[pl.ds(NSUB * RPT, REM)],
                        out_hbm.at[pl.ds(c * N + NSUB * RPT, REM)])


def _sc_segment_sum(h, src, dst, zeros):
    mesh = plsc.VectorSubcoreMesh(core_axis_name="c", subcore_axis_name="s")
    f = pl.kernel(
        _sc_body,
        out_type=jax.ShapeDtypeStruct((2 * N, D), jnp.float32),
        mesh=mesh,
        scratch_types=(
            [pltpu.VMEM((CHUNK,), jnp.int32)] * DEPTH
            + [pltpu.VMEM((CHUNK,), jnp.int32)] * DEPTH
            + [pltpu.VMEM((CHUNK, D), jnp.float32)] * DEPTH
            + [pltpu.VMEM_SHARED((N, D), jnp.float32)]
            + [pltpu.SemaphoreType.DMA] * (4 * DEPTH)
        ),
    )
    return f(h, src, dst, zeros)


def _mlp_phase0(i, p_ref, wa_ref, ba_ref, wb_ref, bb_ref,
                rbuf_ref, a0_ref, a1_ref):
    hin = p_ref[0] + p_ref[1]
    t = jnp.maximum(
        jnp.dot(hin, wa_ref[...], preferred_element_type=jnp.float32,
                precision=_PREC) + ba_ref[...], 0.0)
    r = jnp.maximum(
        jnp.dot(t, wb_ref[...], preferred_element_type=jnp.float32,
                precision=_PREC) + bb_ref[...], 0.0)
    rbuf_ref[pl.ds(i * BLK, BLK), :] = r

    @pl.when(i == 0)
    def _():
        a0_ref[...] = jnp.zeros_like(a0_ref)
        a1_ref[...] = jnp.zeros_like(a1_ref)

    a0_ref[...] += jnp.sum(r, axis=0, keepdims=True)
    a1_ref[...] += jnp.sum(r * r, axis=0, keepdims=True)


def _bn_block(j, rbuf_ref, a0_ref, a1_ref, g_ref, be_ref):
    m = a0_ref[...] * (1.0 / N)
    v = a1_ref[...] * (1.0 / N) - m * m
    inv = lax.rsqrt(v + 1e-5) * g_ref[...]
    return (rbuf_ref[pl.ds(j * BLK, BLK), :] - m) * inv + be_ref[...]


def _layer_body(p_ref, wa_ref, ba_ref, wb_ref, bb_ref,
                g_ref, be_ref, o_ref, rbuf_ref, a0_ref, a1_ref):
    i = pl.program_id(0)

    @pl.when(i < NBLK)
    def _():
        _mlp_phase0(i, p_ref, wa_ref, ba_ref, wb_ref,
                    bb_ref, rbuf_ref, a0_ref, a1_ref)

    @pl.when(i >= NBLK)
    def _():
        o_ref[...] = _bn_block(i - NBLK, rbuf_ref, a0_ref, a1_ref,
                               g_ref, be_ref)


def _tc_layer(parts, wa, ba, wb, bb, g, be):
    full = lambda i: (0, 0)
    return pl.pallas_call(
        _layer_body,
        grid=(2 * NBLK,),
        in_specs=[
            pl.BlockSpec((2, BLK, D),
                         lambda i: (0, jnp.minimum(i, NBLK - 1), 0)),
            pl.BlockSpec((D, D), full),
            pl.BlockSpec((1, D), full),
            pl.BlockSpec((D, D), full),
            pl.BlockSpec((1, D), full),
            pl.BlockSpec((1, D), full),
            pl.BlockSpec((1, D), full),
        ],
        out_specs=pl.BlockSpec((BLK, D), lambda i: (jnp.maximum(i - NBLK,
                                                                0), 0)),
        out_shape=jax.ShapeDtypeStruct((N, D), jnp.float32),
        scratch_shapes=[
            pltpu.VMEM((N, D), jnp.float32),
            pltpu.VMEM((1, D), jnp.float32),
            pltpu.VMEM((1, D), jnp.float32),
        ],
        compiler_params=pltpu.CompilerParams(
            dimension_semantics=("arbitrary",)),
    )(parts.reshape(2, N, D), wa, ba, wb, bb, g, be)


def _final_body(p_ref, b_ref, wa_ref, ba_ref, wb_ref,
                bb_ref, g_ref, be_ref, wf1_ref, bf1_ref, wf2_ref, bf2_ref,
                o_ref, rbuf_ref, a0_ref, a1_ref, pacc_ref):
    i = pl.program_id(0)

    @pl.when(i < NBLK)
    def _():
        _mlp_phase0(i, p_ref, wa_ref, ba_ref, wb_ref,
                    bb_ref, rbuf_ref, a0_ref, a1_ref)

    @pl.when(i >= NBLK)
    def _():
        j = i - NBLK
        hn = _bn_block(j, rbuf_ref, a0_ref, a1_ref, g_ref, be_ref)
        gids = lax.broadcasted_iota(jnp.int32, (1, G), 1).astype(jnp.float32)
        onehot = (b_ref[...] == gids).astype(jnp.float32)

        @pl.when(j == 0)
        def _():
            pacc_ref[...] = jnp.zeros_like(pacc_ref)

        pacc_ref[...] += lax.dot_general(
            onehot, hn, (((0,), (0,)), ((), ())),
            preferred_element_type=jnp.float32, precision=_PREC)

        @pl.when(j == NBLK - 1)
        def _():
            p = jnp.maximum(
                jnp.dot(pacc_ref[...], wf1_ref[...],
                        preferred_element_type=jnp.float32,
                        precision=_PREC) + bf1_ref[...], 0.0)
            o_ref[...] = jnp.dot(p, wf2_ref[...],
                                 preferred_element_type=jnp.float32,
                                 precision=_PREC) + bf2_ref[...]


def _tc_final(parts, batchf, wa, ba, wb, bb, g, be, wf1, bf1, wf2, bf2):
    full = lambda i: (0, 0)
    return pl.pallas_call(
        _final_body,
        grid=(2 * NBLK,),
        in_specs=[
            pl.BlockSpec((2, BLK, D),
                         lambda i: (0, jnp.minimum(i, NBLK - 1), 0)),
            pl.BlockSpec((BLK, 1), lambda i: (jnp.maximum(i - NBLK, 0), 0)),
            pl.BlockSpec((D, D), full),
            pl.BlockSpec((1, D), full),
            pl.BlockSpec((D, D), full),
            pl.BlockSpec((1, D), full),
            pl.BlockSpec((1, D), full),
            pl.BlockSpec((1, D), full),
            pl.BlockSpec((D, D), full),
            pl.BlockSpec((1, D), full),
            pl.BlockSpec((D, C), full),
            pl.BlockSpec((1, C), full),
        ],
        out_specs=pl.BlockSpec((G, C), full),
        out_shape=jax.ShapeDtypeStruct((G, C), jnp.float32),
        scratch_shapes=[
            pltpu.VMEM((N, D), jnp.float32),
            pltpu.VMEM((1, D), jnp.float32),
            pltpu.VMEM((1, D), jnp.float32),
            pltpu.VMEM((G, D), jnp.float32),
        ],
        compiler_params=pltpu.CompilerParams(
            dimension_semantics=("arbitrary",)),
    )(parts.reshape(2, N, D), batchf, wa, ba, wb, bb, g, be,
      wf1, bf1, wf2, bf2)


def _layer(h, src, dst, zeros, wa, ba, wb, bb, g, be):
    parts = _sc_segment_sum(h, src, dst, zeros)
    return _tc_layer(parts, wa, ba, wb, bb, g, be)


def kernel(x, edge_index, batch, w1a, b1a, w1b, b1b, g1, be1,
           w2a, b2a, w2b, b2b, g2, be2, w3a, b3a, w3b, b3b, g3, be3,
           wf1, bf1, wf2, bf2):
    src = edge_index[0]
    dst = edge_index[1]
    zeros = jnp.zeros((N, D), jnp.float32)
    row = lambda v: v.reshape(1, -1)

    h = _layer(x, src, dst, zeros, w1a, row(b1a), w1b, row(b1b),
               row(g1), row(be1))
    h = _layer(h, src, dst, zeros, w2a, row(b2a), w2b, row(b2b),
               row(g2), row(be2))

    batchf = batch.astype(jnp.float32).reshape(N, 1)
    parts3 = _sc_segment_sum(h, src, dst, zeros)
    return _tc_final(parts3, batchf, w3a, row(b3a), w3b, row(b3b),
                     row(g3), row(be3), wf1, row(bf1), wf2, row(bf2))
